# baseline (speedup 1.0000x reference)
import functools

import jax
import jax.numpy as jnp
from jax import lax
from jax.experimental import pallas as pl
from jax.experimental.pallas import tpu as pltpu
from jax.experimental.pallas import tpu_sc as plsc

N = 10000
E = 320000
D_IN = 128
D_OUT = 16

NC = 2
NS = 16
NW = NC * NS
CH = 128
NCHUNK = 80
E_PAD = NW * NCHUNK * CH
NP = 10240
RPT = NP // NS

NQ = 4
QC = NCHUNK // NQ
WIN = 20

_mesh = plsc.VectorSubcoreMesh(core_axis_name="c", subcore_axis_name="s")
_sc_params = pltpu.CompilerParams(use_tc_tiling_on_sc=False)


@functools.partial(
    pl.kernel,
    mesh=_mesh,
    out_type=jax.ShapeDtypeStruct((NC, NP, D_OUT), jnp.float32),
    scratch_types=[
        pltpu.VMEM((NCHUNK, CH), jnp.int32),
        pltpu.VMEM((CH, D_OUT), jnp.float32),
        pltpu.VMEM_SHARED((NP, D_OUT), jnp.float32),
        pltpu.SemaphoreType.DMA,
    ],
    compiler_params=_sc_params,
)
def _deg_pass(col_hbm, ones_hbm, zeros_hbm, deg_hbm, cidx_v, one_v, deg_sh, sem):
    c = lax.axis_index("c")
    s = lax.axis_index("s")
    wid = c * NS + s
    pltpu.sync_copy(ones_hbm, one_v)
    pltpu.sync_copy(col_hbm.at[wid], cidx_v)
    pltpu.sync_copy(zeros_hbm, deg_sh.at[pl.ds(s * RPT, RPT)])
    plsc.subcore_barrier()

    def fire(j):
        pltpu.async_copy(one_v, deg_sh.at[cidx_v.at[j]], sem, add=True)

    def wait_one():
        pltpu.make_async_copy(one_v, deg_sh.at[cidx_v.at[0]], sem).wait()

    def prol(j, carry):
        fire(j)
        return carry

    lax.fori_loop(0, WIN, prol, 0)

    def steady(j, carry):
        wait_one()
        fire(j + WIN)
        return carry

    lax.fori_loop(0, NCHUNK - WIN, steady, 0)

    def drain(j, carry):
        wait_one()
        return carry

    lax.fori_loop(0, WIN, drain, 0)
    plsc.subcore_barrier()
    pltpu.sync_copy(
        deg_sh.at[pl.ds(s * RPT, RPT)], deg_hbm.at[c, pl.ds(s * RPT, RPT)]
    )


@functools.partial(
    pl.kernel,
    mesh=_mesh,
    out_type=jax.ShapeDtypeStruct((NC, NP, D_OUT), jnp.float32),
    scratch_types=[
        pltpu.VMEM((NCHUNK, CH), jnp.int32),
        pltpu.VMEM((NCHUNK, CH), jnp.int32),
        pltpu.VMEM((QC, CH, D_OUT), jnp.float32),
        pltpu.VMEM((QC, CH, D_OUT), jnp.float32),
        pltpu.VMEM_SHARED((NP, D_OUT), jnp.float32),
        pltpu.SemaphoreType.DMA,
        pltpu.SemaphoreType.DMA,
        pltpu.SemaphoreType.DMA,
        pltpu.SemaphoreType.DMA,
    ],
    compiler_params=_sc_params,
)
def _edge_pass(y_hbm, row_hbm, col_hbm, zeros_hbm, acc_hbm,
               ridx_v, cidx_v, buf_a, buf_b, acc_sh, gs_a, gs_b, ss_a, ss_b):
    c = lax.axis_index("c")
    s = lax.axis_index("s")
    wid = c * NS + s
    pltpu.sync_copy(row_hbm.at[wid], ridx_v)
    pltpu.sync_copy(col_hbm.at[wid], cidx_v)
    pltpu.sync_copy(zeros_hbm, acc_sh.at[pl.ds(s * RPT, RPT)])
    plsc.subcore_barrier()

    def fire_g(base, buf, sem):
        def f(k, carry):
            pltpu.async_copy(y_hbm.at[ridx_v.at[base + k]], buf.at[k], sem)
            return carry
        lax.fori_loop(0, QC, f, 0)

    def drain_g(buf, sem):
        def f(k, carry):
            pltpu.make_async_copy(
                y_hbm.at[ridx_v.at[0]], buf.at[0], sem).wait()
            return carry
        lax.fori_loop(0, QC, f, 0)

    def fire_s(base, buf, sem):
        def f(k, carry):
            pltpu.async_copy(
                buf.at[k], acc_sh.at[cidx_v.at[base + k]], sem, add=True)
            return carry
        lax.fori_loop(0, QC, f, 0)

    def drain_s(buf, sem):
        def f(k, carry):
            pltpu.make_async_copy(
                buf.at[0], acc_sh.at[cidx_v.at[0]], sem).wait()
            return carry
        lax.fori_loop(0, QC, f, 0)

    fire_g(0 * QC, buf_a, gs_a)
    fire_g(1 * QC, buf_b, gs_b)
    drain_g(buf_a, gs_a)
    fire_s(0 * QC, buf_a, ss_a)
    drain_g(buf_b, gs_b)
    fire_s(1 * QC, buf_b, ss_b)
    drain_s(buf_a, ss_a)
    fire_g(2 * QC, buf_a, gs_a)
    drain_g(buf_a, gs_a)
    fire_s(2 * QC, buf_a, ss_a)
    drain_s(buf_b, ss_b)
    fire_g(3 * QC, buf_b, gs_b)
    drain_g(buf_b, gs_b)
    fire_s(3 * QC, buf_b, ss_b)
    drain_s(buf_a, ss_a)
    drain_s(buf_b, ss_b)

    plsc.subcore_barrier()
    pltpu.sync_copy(
        acc_sh.at[pl.ds(s * RPT, RPT)], acc_hbm.at[c, pl.ds(s * RPT, RPT)]
    )


def _xw_body(x_ref, w_ref, deg_ref, y_ref, dis_ref):
    deg = deg_ref[0, :N, :] + deg_ref[1, :N, :] + 1.0
    dis = lax.rsqrt(deg)
    xw = jnp.dot(x_ref[...], w_ref[...], preferred_element_type=jnp.float32)
    y_ref[...] = xw * dis
    dis_ref[...] = dis


def _xw_call(x, W, deg_parts):
    return pl.pallas_call(
        _xw_body,
        out_shape=[
            jax.ShapeDtypeStruct((N, D_OUT), jnp.float32),
            jax.ShapeDtypeStruct((N, D_OUT), jnp.float32),
        ],
    )(x, W, deg_parts)


def _fin_body(acc_ref, y_ref, dis_ref, b_ref, out_ref):
    t = (acc_ref[0, :N, :] + acc_ref[1, :N, :] + y_ref[...]) * dis_ref[...]
    t = t + b_ref[...]
    m = jnp.max(t, axis=1, keepdims=True)
    ls = jnp.log(jnp.sum(jnp.exp(t - m), axis=1, keepdims=True))
    out_ref[...] = t - m - ls


def _fin_call(acc_parts, y, dis, b2d):
    return pl.pallas_call(
        _fin_body,
        out_shape=jax.ShapeDtypeStruct((N, D_OUT), jnp.float32),
    )(acc_parts, y, dis, b2d)


@jax.jit
def kernel(x, edge_index, W, b):
    row = edge_index[0]
    col = edge_index[1]
    pad = E_PAD - E
    rowp = jnp.concatenate(
        [row, jnp.zeros((pad,), jnp.int32)]).reshape(NW, NCHUNK, CH)
    colp = jnp.concatenate(
        [col, jnp.full((pad,), N, jnp.int32)]).reshape(NW, NCHUNK, CH)

    ones_rows = jnp.ones((CH, D_OUT), jnp.float32)
    zeros_rows = jnp.zeros((RPT, D_OUT), jnp.float32)

    deg_parts = _deg_pass(colp, ones_rows, zeros_rows)
    y, dis = _xw_call(x, W, deg_parts)
    acc_parts = _edge_pass(y, rowp, colp, zeros_rows)
    return _fin_call(acc_parts, y, dis, b.reshape(1, D_OUT))


# speedup vs baseline: 1.0606x; 1.0606x over previous
import functools

import jax
import jax.numpy as jnp
from jax import lax
from jax.experimental import pallas as pl
from jax.experimental.pallas import tpu as pltpu
from jax.experimental.pallas import tpu_sc as plsc

N = 10000
E = 320000
D_IN = 128
D_OUT = 16

NC = 2
NS = 16
NW = NC * NS
CH = 128

K0 = 112
K1 = 48
TOT = NS * (K0 + K1)
TOT_PAD = TOT + K0
E_PAD = TOT * CH

NP = 10240
RPT = NP // NS

NB = 8
WIN = 20

_mesh = plsc.VectorSubcoreMesh(core_axis_name="c", subcore_axis_name="s")
_sc_params = pltpu.CompilerParams(use_tc_tiling_on_sc=False)


def _tile_span(c, s):
    k = jnp.where(c == 0, K0, K1)
    base = jnp.where(c == 0, s * K0, NS * K0 + s * K1)
    ngrp = jnp.where(c == 0, K0 // NB, K1 // NB)
    return base, k, ngrp


@functools.partial(
    pl.kernel,
    mesh=_mesh,
    out_type=jax.ShapeDtypeStruct((NC, NP, D_OUT), jnp.float32),
    scratch_types=[
        pltpu.VMEM((K0, CH), jnp.int32),
        pltpu.VMEM((CH, D_OUT), jnp.float32),
        pltpu.VMEM_SHARED((NP, D_OUT), jnp.float32),
        pltpu.SemaphoreType.DMA,
    ],
    compiler_params=_sc_params,
)
def _deg_pass(col_hbm, ones_hbm, zeros_hbm, deg_hbm, cidx_v, one_v, deg_sh, sem):
    c = lax.axis_index("c")
    s = lax.axis_index("s")
    base, k, _ = _tile_span(c, s)
    pltpu.sync_copy(ones_hbm, one_v)
    pltpu.sync_copy(col_hbm.at[pl.ds(base, K0)], cidx_v)
    pltpu.sync_copy(zeros_hbm, deg_sh.at[pl.ds(s * RPT, RPT)])
    plsc.subcore_barrier()

    def fire(j):
        pltpu.async_copy(one_v, deg_sh.at[cidx_v.at[j]], sem, add=True)

    def wait_one():
        pltpu.make_async_copy(one_v, deg_sh.at[cidx_v.at[0]], sem).wait()

    def prol(j, carry):
        fire(j)
        return carry

    lax.fori_loop(0, WIN, prol, 0)

    def steady(j, carry):
        wait_one()
        fire(j + WIN)
        return carry

    lax.fori_loop(0, k - WIN, steady, 0)

    def drain(j, carry):
        wait_one()
        return carry

    lax.fori_loop(0, WIN, drain, 0)
    plsc.subcore_barrier()
    pltpu.sync_copy(
        deg_sh.at[pl.ds(s * RPT, RPT)], deg_hbm.at[c, pl.ds(s * RPT, RPT)]
    )


@functools.partial(
    pl.kernel,
    mesh=_mesh,
    out_type=jax.ShapeDtypeStruct((NC, NP, D_OUT), jnp.float32),
    scratch_types=(
        [
            pltpu.VMEM((K0, CH), jnp.int32),
            pltpu.VMEM((K0, CH), jnp.int32),
            pltpu.VMEM((NB, CH, D_OUT), jnp.float32),
            pltpu.VMEM_SHARED((NP, D_OUT), jnp.float32),
        ]
        + [pltpu.SemaphoreType.DMA] * (2 * NB)
    ),
    compiler_params=_sc_params,
)
def _edge_pass(y_hbm, row_hbm, col_hbm, zeros_hbm, acc_hbm,
               ridx_v, cidx_v, rows_v, acc_sh, *sems):
    gsem = sems[:NB]
    ssem = sems[NB:]
    c = lax.axis_index("c")
    s = lax.axis_index("s")
    base, k, ngrp = _tile_span(c, s)
    pltpu.sync_copy(row_hbm.at[pl.ds(base, K0)], ridx_v)
    pltpu.sync_copy(col_hbm.at[pl.ds(base, K0)], cidx_v)
    pltpu.sync_copy(zeros_hbm, acc_sh.at[pl.ds(s * RPT, RPT)])
    plsc.subcore_barrier()

    for b in range(NB):
        pltpu.async_copy(y_hbm.at[ridx_v.at[b]], rows_v.at[b], gsem[b])

    def group(jo, carry):
        for b in range(NB):
            j = jo * NB + b
            pltpu.make_async_copy(
                y_hbm.at[ridx_v.at[j]], rows_v.at[b], gsem[b]).wait()
            pltpu.async_copy(
                rows_v.at[b], acc_sh.at[cidx_v.at[j]], ssem[b], add=True)
            pltpu.make_async_copy(
                rows_v.at[b], acc_sh.at[cidx_v.at[j]], ssem[b]).wait()
            pltpu.async_copy(
                y_hbm.at[ridx_v.at[j + NB]], rows_v.at[b], gsem[b])
        return carry

    lax.fori_loop(0, ngrp - 1, group, 0)

    for b in range(NB):
        j = (ngrp - 1) * NB + b
        pltpu.make_async_copy(
            y_hbm.at[ridx_v.at[j]], rows_v.at[b], gsem[b]).wait()
        pltpu.async_copy(
            rows_v.at[b], acc_sh.at[cidx_v.at[j]], ssem[b], add=True)
    for b in range(NB):
        pltpu.make_async_copy(
            rows_v.at[b], acc_sh.at[cidx_v.at[0]], ssem[b]).wait()

    plsc.subcore_barrier()
    pltpu.sync_copy(
        acc_sh.at[pl.ds(s * RPT, RPT)], acc_hbm.at[c, pl.ds(s * RPT, RPT)]
    )


def _xw_body(x_ref, w_ref, deg_ref, y_ref, dis_ref):
    deg = deg_ref[0, :N, :] + deg_ref[1, :N, :] + 1.0
    dis = lax.rsqrt(deg)
    xw = jnp.dot(x_ref[...], w_ref[...], preferred_element_type=jnp.float32)
    y_ref[...] = xw * dis
    dis_ref[...] = dis


def _xw_call(x, W, deg_parts):
    return pl.pallas_call(
        _xw_body,
        out_shape=[
            jax.ShapeDtypeStruct((N, D_OUT), jnp.float32),
            jax.ShapeDtypeStruct((N, D_OUT), jnp.float32),
        ],
    )(x, W, deg_parts)


def _fin_body(acc_ref, y_ref, dis_ref, b_ref, out_ref):
    t = (acc_ref[0, :N, :] + acc_ref[1, :N, :] + y_ref[...]) * dis_ref[...]
    t = t + b_ref[...]
    m = jnp.max(t, axis=1, keepdims=True)
    ls = jnp.log(jnp.sum(jnp.exp(t - m), axis=1, keepdims=True))
    out_ref[...] = t - m - ls


def _fin_call(acc_parts, y, dis, b2d):
    return pl.pallas_call(
        _fin_body,
        out_shape=jax.ShapeDtypeStruct((N, D_OUT), jnp.float32),
    )(acc_parts, y, dis, b2d)


@jax.jit
def kernel(x, edge_index, W, b):
    row = edge_index[0]
    col = edge_index[1]
    pad = TOT_PAD * CH - E
    rowp = jnp.concatenate(
        [row, jnp.zeros((pad,), jnp.int32)]).reshape(TOT_PAD, CH)
    colp = jnp.concatenate(
        [col, jnp.full((pad,), N, jnp.int32)]).reshape(TOT_PAD, CH)

    ones_rows = jnp.ones((CH, D_OUT), jnp.float32)
    zeros_rows = jnp.zeros((RPT, D_OUT), jnp.float32)

    deg_parts = _deg_pass(colp, ones_rows, zeros_rows)
    y, dis = _xw_call(x, W, deg_parts)
    acc_parts = _edge_pass(y, rowp, colp, zeros_rows)
    return _fin_call(acc_parts, y, dis, b.reshape(1, D_OUT))


# speedup vs baseline: 1.3744x; 1.2959x over previous
import functools

import jax
import jax.numpy as jnp
from jax import lax
from jax.experimental import pallas as pl
from jax.experimental.pallas import tpu as pltpu
from jax.experimental.pallas import tpu_sc as plsc

N = 10000
E = 320000
D_IN = 128
D_OUT = 16

NC = 2
NS = 16
NW = NC * NS
CH = 128

K0 = 80
K1 = 80
TOT = NS * (K0 + K1)
TOT_PAD = TOT + K0
E_PAD = TOT * CH

NP = 10240
RPT = NP // NS

NB = 8
WIN = 20

_mesh = plsc.VectorSubcoreMesh(core_axis_name="c", subcore_axis_name="s")
_sc_params = pltpu.CompilerParams(use_tc_tiling_on_sc=False)


def _tile_span(c, s):
    k = jnp.where(c == 0, K0, K1)
    base = jnp.where(c == 0, s * K0, NS * K0 + s * K1)
    ngrp = jnp.where(c == 0, K0 // NB, K1 // NB)
    return base, k, ngrp


@functools.partial(
    pl.kernel,
    mesh=_mesh,
    out_type=jax.ShapeDtypeStruct((NC, NP, D_OUT), jnp.float32),
    scratch_types=[
        pltpu.VMEM((K0, CH), jnp.int32),
        pltpu.VMEM((CH, D_OUT), jnp.float32),
        pltpu.VMEM_SHARED((NP, D_OUT), jnp.float32),
        pltpu.SemaphoreType.DMA,
    ],
    compiler_params=_sc_params,
)
def _deg_pass(col_hbm, ones_hbm, zeros_hbm, deg_hbm, cidx_v, one_v, deg_sh, sem):
    c = lax.axis_index("c")
    s = lax.axis_index("s")
    base, k, _ = _tile_span(c, s)
    pltpu.sync_copy(ones_hbm, one_v)
    pltpu.sync_copy(col_hbm.at[pl.ds(base, K0)], cidx_v)
    pltpu.sync_copy(zeros_hbm, deg_sh.at[pl.ds(s * RPT, RPT)])
    plsc.subcore_barrier()

    def fire(j):
        pltpu.async_copy(one_v, deg_sh.at[cidx_v.at[j]], sem, add=True)

    def wait_one():
        pltpu.make_async_copy(one_v, deg_sh.at[cidx_v.at[0]], sem).wait()

    def prol(j, carry):
        fire(j)
        return carry

    lax.fori_loop(0, WIN, prol, 0)

    def steady(j, carry):
        wait_one()
        fire(j + WIN)
        return carry

    lax.fori_loop(0, k - WIN, steady, 0)

    def drain(j, carry):
        wait_one()
        return carry

    lax.fori_loop(0, WIN, drain, 0)
    plsc.subcore_barrier()
    pltpu.sync_copy(
        deg_sh.at[pl.ds(s * RPT, RPT)], deg_hbm.at[c, pl.ds(s * RPT, RPT)]
    )


@functools.partial(
    pl.kernel,
    mesh=_mesh,
    out_type=jax.ShapeDtypeStruct((NC, NP, D_OUT), jnp.float32),
    scratch_types=(
        [
            pltpu.VMEM((K0, CH), jnp.int32),
            pltpu.VMEM((K0, CH), jnp.int32),
            pltpu.VMEM((NB, CH, D_OUT), jnp.float32),
            pltpu.VMEM_SHARED((NP, D_OUT), jnp.float32),
            pltpu.VMEM_SHARED((N, D_OUT), jnp.float32),
        ]
        + [pltpu.SemaphoreType.DMA] * (2 * NB)
    ),
    compiler_params=_sc_params,
)
def _edge_pass(y_hbm, row_hbm, col_hbm, zeros_hbm, acc_hbm,
               ridx_v, cidx_v, rows_v, acc_sh, y_sh, *sems):
    gsem = sems[:NB]
    ssem = sems[NB:]
    c = lax.axis_index("c")
    s = lax.axis_index("s")
    base, k, ngrp = _tile_span(c, s)
    YR = N // NS
    pltpu.sync_copy(y_hbm.at[pl.ds(s * YR, YR)], y_sh.at[pl.ds(s * YR, YR)])
    pltpu.sync_copy(row_hbm.at[pl.ds(base, K0)], ridx_v)
    pltpu.sync_copy(col_hbm.at[pl.ds(base, K0)], cidx_v)
    pltpu.sync_copy(zeros_hbm, acc_sh.at[pl.ds(s * RPT, RPT)])
    plsc.subcore_barrier()

    for b in range(NB):
        pltpu.async_copy(y_sh.at[ridx_v.at[b]], rows_v.at[b], gsem[b])

    def group(jo, carry):
        for b in range(NB):
            j = jo * NB + b
            pltpu.make_async_copy(
                y_sh.at[ridx_v.at[j]], rows_v.at[b], gsem[b]).wait()
            pltpu.async_copy(
                rows_v.at[b], acc_sh.at[cidx_v.at[j]], ssem[b], add=True)
            pltpu.make_async_copy(
                rows_v.at[b], acc_sh.at[cidx_v.at[j]], ssem[b]).wait()
            pltpu.async_copy(
                y_sh.at[ridx_v.at[j + NB]], rows_v.at[b], gsem[b])
        return carry

    lax.fori_loop(0, ngrp - 1, group, 0)

    for b in range(NB):
        j = (ngrp - 1) * NB + b
        pltpu.make_async_copy(
            y_sh.at[ridx_v.at[j]], rows_v.at[b], gsem[b]).wait()
        pltpu.async_copy(
            rows_v.at[b], acc_sh.at[cidx_v.at[j]], ssem[b], add=True)
    for b in range(NB):
        pltpu.make_async_copy(
            rows_v.at[b], acc_sh.at[cidx_v.at[0]], ssem[b]).wait()

    plsc.subcore_barrier()
    pltpu.sync_copy(
        acc_sh.at[pl.ds(s * RPT, RPT)], acc_hbm.at[c, pl.ds(s * RPT, RPT)]
    )


def _xw_body(x_ref, w_ref, deg_ref, y_ref, dis_ref):
    deg = deg_ref[0, :N, :] + deg_ref[1, :N, :] + 1.0
    dis = lax.rsqrt(deg)
    xw = jnp.dot(x_ref[...], w_ref[...], preferred_element_type=jnp.float32)
    y_ref[...] = xw * dis
    dis_ref[...] = dis


def _xw_call(x, W, deg_parts):
    return pl.pallas_call(
        _xw_body,
        out_shape=[
            jax.ShapeDtypeStruct((N, D_OUT), jnp.float32),
            jax.ShapeDtypeStruct((N, D_OUT), jnp.float32),
        ],
    )(x, W, deg_parts)


def _fin_body(acc_ref, y_ref, dis_ref, b_ref, out_ref):
    t = (acc_ref[0, :N, :] + acc_ref[1, :N, :] + y_ref[...]) * dis_ref[...]
    t = t + b_ref[...]
    m = jnp.max(t, axis=1, keepdims=True)
    ls = jnp.log(jnp.sum(jnp.exp(t - m), axis=1, keepdims=True))
    out_ref[...] = t - m - ls


def _fin_call(acc_parts, y, dis, b2d):
    return pl.pallas_call(
        _fin_body,
        out_shape=jax.ShapeDtypeStruct((N, D_OUT), jnp.float32),
    )(acc_parts, y, dis, b2d)


@jax.jit
def kernel(x, edge_index, W, b):
    row = edge_index[0]
    col = edge_index[1]
    pad = TOT_PAD * CH - E
    rowp = jnp.concatenate(
        [row, jnp.zeros((pad,), jnp.int32)]).reshape(TOT_PAD, CH)
    colp = jnp.concatenate(
        [col, jnp.full((pad,), N, jnp.int32)]).reshape(TOT_PAD, CH)

    ones_rows = jnp.ones((CH, D_OUT), jnp.float32)
    zeros_rows = jnp.zeros((RPT, D_OUT), jnp.float32)

    deg_parts = _deg_pass(colp, ones_rows, zeros_rows)
    y, dis = _xw_call(x, W, deg_parts)
    acc_parts = _edge_pass(y, rowp, colp, zeros_rows)
    return _fin_call(acc_parts, y, dis, b.reshape(1, D_OUT))


# speedup vs baseline: 1.4768x; 1.0746x over previous
import functools

import jax
import jax.numpy as jnp
from jax import lax
from jax.experimental import pallas as pl
from jax.experimental.pallas import tpu as pltpu
from jax.experimental.pallas import tpu_sc as plsc

N = 10000
E = 320000
D_IN = 128
D_OUT = 16

NC = 2
NS = 16
CH = 128

KD = 160
KE = 80
TOT = NS * KD
E_PAD = TOT * CH

NP = 10240
RPT = NP // NS

NB = 8
NGRP = KE // NB
WIN = 20

_mesh = plsc.VectorSubcoreMesh(core_axis_name="c", subcore_axis_name="s")
_sc_params = pltpu.CompilerParams(use_tc_tiling_on_sc=False)


@functools.partial(
    pl.kernel,
    mesh=_mesh,
    out_type=[
        jax.ShapeDtypeStruct((NP,), jnp.float32),
        jax.ShapeDtypeStruct((NC, NP, D_OUT), jnp.float32),
    ],
    scratch_types=(
        [
            pltpu.VMEM((KD, CH), jnp.int32),
            pltpu.VMEM((KE, CH), jnp.int32),
            pltpu.VMEM((CH,), jnp.float32),
            pltpu.VMEM((RPT,), jnp.float32),
            pltpu.VMEM((RPT, D_OUT), jnp.float32),
            pltpu.VMEM((NB, CH, D_OUT), jnp.float32),
            pltpu.VMEM_SHARED((NP,), jnp.float32),
            pltpu.VMEM_SHARED((NP, D_OUT), jnp.float32),
            pltpu.VMEM_SHARED((NP, D_OUT), jnp.float32),
        ]
        + [pltpu.SemaphoreType.DMA] * (1 + 2 * NB)
    ),
    compiler_params=_sc_params,
)
def _sc_mega(xw_hbm, row_hbm, col_hbm, ones_hbm, zer1_hbm, zer16_hbm,
             deg_hbm, acc_hbm,
             cidx_v, ridx_v, one_v, dbuf, xbuf, rows_v,
             deg_sh, y_sh, acc_sh, *sems):
    dsem = sems[0]
    gsem = sems[1:1 + NB]
    ssem = sems[1 + NB:]
    c = lax.axis_index("c")
    s = lax.axis_index("s")

    pltpu.sync_copy(ones_hbm, one_v)
    pltpu.sync_copy(col_hbm.at[pl.ds(s * KD, KD)], cidx_v)
    pltpu.sync_copy(row_hbm.at[pl.ds(s * KD + c * KE, KE)], ridx_v)
    pltpu.sync_copy(zer1_hbm, deg_sh.at[pl.ds(s * RPT, RPT)])
    pltpu.sync_copy(zer16_hbm, acc_sh.at[pl.ds(s * RPT, RPT)])
    plsc.subcore_barrier()

    def dfire(j):
        pltpu.async_copy(one_v, deg_sh.at[cidx_v.at[j]], dsem, add=True)

    def dwait():
        pltpu.make_async_copy(one_v, deg_sh.at[cidx_v.at[0]], dsem).wait()

    def dprol(j, carry):
        dfire(j)
        return carry

    lax.fori_loop(0, WIN, dprol, 0)

    def dsteady(j, carry):
        dwait()
        dfire(j + WIN)
        return carry

    lax.fori_loop(0, KD - WIN, dsteady, 0)

    def ddrain(j, carry):
        dwait()
        return carry

    lax.fori_loop(0, WIN, ddrain, 0)
    plsc.subcore_barrier()

    pltpu.sync_copy(deg_sh.at[pl.ds(s * RPT, RPT)], dbuf)
    @pl.when(c == 0)
    def _():
        pltpu.sync_copy(dbuf, deg_hbm.at[pl.ds(s * RPT, RPT)])
    pltpu.sync_copy(xw_hbm.at[pl.ds(s * RPT, RPT)], xbuf)

    def yrow(i, carry):
        d = dbuf[pl.ds(i * 16, 16)] + 1.0
        ib = lax.bitcast_convert_type(d, jnp.int32)
        ib = jnp.int32(0x5F3759DF) - lax.shift_right_arithmetic(ib, 1)
        r = lax.bitcast_convert_type(ib, jnp.float32)
        r = r * (1.5 - 0.5 * d * r * r)
        r = r * (1.5 - 0.5 * d * r * r)
        r = r * (1.5 - 0.5 * d * r * r)
        for t in range(16):
            rt = jax.lax.gather(
                r,
                jnp.full((16, 1), t, jnp.int32),
                jax.lax.GatherDimensionNumbers(
                    offset_dims=(), collapsed_slice_dims=(0,),
                    start_index_map=(0,)),
                (1,),
                mode=jax.lax.GatherScatterMode.PROMISE_IN_BOUNDS,
            )
            row = i * 16 + t
            xbuf[row, :] = xbuf[row, :] * rt
        return carry

    lax.fori_loop(0, RPT // 16, yrow, 0)
    pltpu.sync_copy(xbuf, y_sh.at[pl.ds(s * RPT, RPT)])
    plsc.subcore_barrier()

    ebase = c * KE

    def gsrc(j):
        return y_sh.at[ridx_v.at[j]]

    def sdst(j):
        return acc_sh.at[cidx_v.at[ebase + j]]

    for b in range(NB):
        pltpu.async_copy(gsrc(b), rows_v.at[b], gsem[b])

    def group(jo, carry):
        for b in range(NB):
            j = jo * NB + b
            pltpu.make_async_copy(gsrc(j), rows_v.at[b], gsem[b]).wait()
            pltpu.async_copy(rows_v.at[b], sdst(j), ssem[b], add=True)
            pltpu.make_async_copy(rows_v.at[b], sdst(j), ssem[b]).wait()
            pltpu.async_copy(gsrc(j + NB), rows_v.at[b], gsem[b])
        return carry

    lax.fori_loop(0, NGRP - 1, group, 0)

    for b in range(NB):
        j = (NGRP - 1) * NB + b
        pltpu.make_async_copy(gsrc(j), rows_v.at[b], gsem[b]).wait()
        pltpu.async_copy(rows_v.at[b], sdst(j), ssem[b], add=True)
    for b in range(NB):
        pltpu.make_async_copy(rows_v.at[b], sdst(0), ssem[b]).wait()

    plsc.subcore_barrier()
    pltpu.sync_copy(
        acc_sh.at[pl.ds(s * RPT, RPT)], acc_hbm.at[c, pl.ds(s * RPT, RPT)]
    )


def _xw_body(x_ref, w_ref, y_ref):
    y_ref[pl.ds(0, N), :] = jnp.dot(
        x_ref[...], w_ref[...], preferred_element_type=jnp.float32)
    y_ref[pl.ds(N, NP - N), :] = jnp.zeros((NP - N, D_OUT), jnp.float32)


def _xw_call(x, W):
    return pl.pallas_call(
        _xw_body,
        out_shape=jax.ShapeDtypeStruct((NP, D_OUT), jnp.float32),
    )(x, W)


def _fin_body(acc_ref, deg_ref, xw_ref, b_ref, out_ref):
    dis = lax.rsqrt(deg_ref[...] + 1.0)
    y = xw_ref[:N, :] * dis
    t = (acc_ref[0, :N, :] + acc_ref[1, :N, :] + y) * dis + b_ref[...]
    m = jnp.max(t, axis=1, keepdims=True)
    ls = jnp.log(jnp.sum(jnp.exp(t - m), axis=1, keepdims=True))
    out_ref[...] = t - m - ls


def _fin_call(acc_parts, deg, xw, b2d):
    return pl.pallas_call(
        _fin_body,
        out_shape=jax.ShapeDtypeStruct((N, D_OUT), jnp.float32),
    )(acc_parts, deg[:N].reshape(N, 1), xw, b2d)


@jax.jit
def kernel(x, edge_index, W, b):
    row = edge_index[0]
    col = edge_index[1]
    pad = E_PAD - E
    rowp = jnp.concatenate(
        [row, jnp.zeros((pad,), jnp.int32)]).reshape(TOT, CH)
    colp = jnp.concatenate(
        [col, jnp.full((pad,), N, jnp.int32)]).reshape(TOT, CH)

    ones_v = jnp.ones((CH,), jnp.float32)
    zer1 = jnp.zeros((RPT,), jnp.float32)
    zer16 = jnp.zeros((RPT, D_OUT), jnp.float32)

    xw = _xw_call(x, W)
    deg, acc_parts = _sc_mega(xw, rowp, colp, ones_v, zer1, zer16)
    return _fin_call(acc_parts, deg, xw, b.reshape(1, D_OUT))


# speedup vs baseline: 1.4895x; 1.0086x over previous
import functools

import jax
import jax.numpy as jnp
from jax import lax
from jax.experimental import pallas as pl
from jax.experimental.pallas import tpu as pltpu
from jax.experimental.pallas import tpu_sc as plsc

N = 10000
E = 320000
D_IN = 128
D_OUT = 16

NC = 2
NS = 16
CH = 128

KD = 160
KE = 80
TOT = NS * KD
E_PAD = TOT * CH

NP = 10240
RPT = NP // NS

NB = 8
NGRP = KE // NB
WIN = 20

_mesh = plsc.VectorSubcoreMesh(core_axis_name="c", subcore_axis_name="s")
_sc_params = pltpu.CompilerParams(use_tc_tiling_on_sc=False)


@functools.partial(
    pl.kernel,
    mesh=_mesh,
    out_type=[
        jax.ShapeDtypeStruct((NP,), jnp.float32),
        jax.ShapeDtypeStruct((NC, NP, D_OUT), jnp.float32),
    ],
    scratch_types=(
        [
            pltpu.VMEM((KD, CH), jnp.int32),
            pltpu.VMEM((KE, CH), jnp.int32),
            pltpu.VMEM((CH,), jnp.float32),
            pltpu.VMEM((RPT,), jnp.float32),
            pltpu.VMEM((RPT, D_OUT), jnp.float32),
            pltpu.VMEM((NB, CH, D_OUT), jnp.float32),
            pltpu.VMEM_SHARED((NP,), jnp.float32),
            pltpu.VMEM_SHARED((NP, D_OUT), jnp.float32),
            pltpu.VMEM_SHARED((NP, D_OUT), jnp.float32),
        ]
        + [pltpu.SemaphoreType.DMA] * (2 + 2 * NB)
    ),
    compiler_params=_sc_params,
)
def _sc_mega(xw_hbm, row_hbm, col_hbm, ones_hbm, zer1_hbm, zer16_hbm,
             deg_hbm, acc_hbm,
             cidx_v, ridx_v, one_v, dbuf, xbuf, rows_v,
             deg_sh, y_sh, acc_sh, *sems):
    dsem = sems[0]
    xsem = sems[1]
    gsem = sems[2:2 + NB]
    ssem = sems[2 + NB:]
    c = lax.axis_index("c")
    s = lax.axis_index("s")

    pltpu.sync_copy(ones_hbm, one_v)
    pltpu.sync_copy(col_hbm.at[pl.ds(s * KD, KD)], cidx_v)
    pltpu.sync_copy(row_hbm.at[pl.ds(s * KD + c * KE, KE)], ridx_v)
    pltpu.sync_copy(zer1_hbm, deg_sh.at[pl.ds(s * RPT, RPT)])
    pltpu.sync_copy(zer16_hbm, acc_sh.at[pl.ds(s * RPT, RPT)])
    pltpu.async_copy(xw_hbm.at[pl.ds(s * RPT, RPT)], xbuf, xsem)
    plsc.subcore_barrier()

    def dfire(j):
        pltpu.async_copy(one_v, deg_sh.at[cidx_v.at[j]], dsem, add=True)

    def dwait():
        pltpu.make_async_copy(one_v, deg_sh.at[cidx_v.at[0]], dsem).wait()

    def dprol(j, carry):
        dfire(j)
        return carry

    lax.fori_loop(0, WIN, dprol, 0)

    def dsteady(j, carry):
        dwait()
        dfire(j + WIN)
        return carry

    lax.fori_loop(0, KD - WIN, dsteady, 0)

    def ddrain(j, carry):
        dwait()
        return carry

    lax.fori_loop(0, WIN, ddrain, 0)
    plsc.subcore_barrier()

    pltpu.sync_copy(deg_sh.at[pl.ds(s * RPT, RPT)], dbuf)
    @pl.when(c == 0)
    def _():
        pltpu.sync_copy(dbuf, deg_hbm.at[pl.ds(s * RPT, RPT)])
    pltpu.make_async_copy(xw_hbm.at[pl.ds(s * RPT, RPT)], xbuf, xsem).wait()

    def yrow(i, carry):
        d = dbuf[pl.ds(i * 16, 16)] + 1.0
        ib = lax.bitcast_convert_type(d, jnp.int32)
        ib = jnp.int32(0x5F3759DF) - lax.shift_right_arithmetic(ib, 1)
        r = lax.bitcast_convert_type(ib, jnp.float32)
        r = r * (1.5 - 0.5 * d * r * r)
        r = r * (1.5 - 0.5 * d * r * r)
        r = r * (1.5 - 0.5 * d * r * r)
        for t in range(16):
            rt = jax.lax.gather(
                r,
                jnp.full((16, 1), t, jnp.int32),
                jax.lax.GatherDimensionNumbers(
                    offset_dims=(), collapsed_slice_dims=(0,),
                    start_index_map=(0,)),
                (1,),
                mode=jax.lax.GatherScatterMode.PROMISE_IN_BOUNDS,
            )
            row = i * 16 + t
            xbuf[row, :] = xbuf[row, :] * rt
        return carry

    lax.fori_loop(0, RPT // 16, yrow, 0)
    pltpu.sync_copy(xbuf, y_sh.at[pl.ds(s * RPT, RPT)])
    plsc.subcore_barrier()

    ebase = c * KE

    def gsrc(j):
        return y_sh.at[ridx_v.at[j]]

    def sdst(j):
        return acc_sh.at[cidx_v.at[ebase + j]]

    for b in range(NB):
        pltpu.async_copy(gsrc(b), rows_v.at[b], gsem[b])

    def group(jo, carry):
        for b in range(NB):
            j = jo * NB + b
            pltpu.make_async_copy(gsrc(j), rows_v.at[b], gsem[b]).wait()
            pltpu.async_copy(rows_v.at[b], sdst(j), ssem[b], add=True)
            pltpu.make_async_copy(rows_v.at[b], sdst(j), ssem[b]).wait()
            pltpu.async_copy(gsrc(j + NB), rows_v.at[b], gsem[b])
        return carry

    lax.fori_loop(0, NGRP - 1, group, 0)

    for b in range(NB):
        j = (NGRP - 1) * NB + b
        pltpu.make_async_copy(gsrc(j), rows_v.at[b], gsem[b]).wait()
        pltpu.async_copy(rows_v.at[b], sdst(j), ssem[b], add=True)
    for b in range(NB):
        pltpu.make_async_copy(rows_v.at[b], sdst(0), ssem[b]).wait()

    plsc.subcore_barrier()
    pltpu.sync_copy(
        acc_sh.at[pl.ds(s * RPT, RPT)], acc_hbm.at[c, pl.ds(s * RPT, RPT)]
    )


def _xw_body(x_ref, w_ref, y_ref):
    y_ref[pl.ds(0, N), :] = jnp.dot(
        x_ref[...], w_ref[...], preferred_element_type=jnp.float32)
    y_ref[pl.ds(N, NP - N), :] = jnp.zeros((NP - N, D_OUT), jnp.float32)


def _xw_call(x, W):
    return pl.pallas_call(
        _xw_body,
        out_shape=jax.ShapeDtypeStruct((NP, D_OUT), jnp.float32),
    )(x, W)


def _fin_body(acc_ref, deg_ref, xw_ref, b_ref, out_ref):
    dis = lax.rsqrt(deg_ref[...] + 1.0)
    y = xw_ref[:N, :] * dis
    t = (acc_ref[0, :N, :] + acc_ref[1, :N, :] + y) * dis + b_ref[...]
    m = jnp.max(t, axis=1, keepdims=True)
    ls = jnp.log(jnp.sum(jnp.exp(t - m), axis=1, keepdims=True))
    out_ref[...] = t - m - ls


def _fin_call(acc_parts, deg, xw, b2d):
    return pl.pallas_call(
        _fin_body,
        out_shape=jax.ShapeDtypeStruct((N, D_OUT), jnp.float32),
    )(acc_parts, deg[:N].reshape(N, 1), xw, b2d)


@jax.jit
def kernel(x, edge_index, W, b):
    row = edge_index[0]
    col = edge_index[1]
    pad = E_PAD - E
    rowp = jnp.concatenate(
        [row, jnp.zeros((pad,), jnp.int32)]).reshape(TOT, CH)
    colp = jnp.concatenate(
        [col, jnp.full((pad,), N, jnp.int32)]).reshape(TOT, CH)

    ones_v = jnp.ones((CH,), jnp.float32)
    zer1 = jnp.zeros((RPT,), jnp.float32)
    zer16 = jnp.zeros((RPT, D_OUT), jnp.float32)

    xw = _xw_call(x, W)
    deg, acc_parts = _sc_mega(xw, rowp, colp, ones_v, zer1, zer16)
    return _fin_call(acc_parts, deg, xw, b.reshape(1, D_OUT))
